# Initial kernel scaffold; baseline (speedup 1.0000x reference)
import jax
import jax.numpy as jnp
from jax.experimental import pallas as pl


def kernel(x, edge_index, W1, S1, b1, W2, S2, b2, Wf, bf):
    raise NotImplementedError("write your pallas kernel here")


# speedup vs baseline: 3.3886x; 3.3886x over previous
import jax
import jax.numpy as jnp
from jax import lax
from jax.experimental import pallas as pl
from jax.experimental.pallas import tpu as pltpu
from jax.experimental.pallas import tpu_sc as plsc

N = 10000
E = 320000
D = 128
DH = D // 2
NC = 2
NS = 16
NW = NC * NS
CHUNK = 128
NCHUNK = 80
E_PAD = NW * NCHUNK * CHUNK
EPW = E_PAD // NW
NT = NCHUNK // 2
NP = 10008
BLK = 1000


def _sc_agg(h_lo, h_hi, cols, rows):
    mesh = plsc.VectorSubcoreMesh(core_axis_name="c", subcore_axis_name="s")

    def body(hlo_hbm, hhi_hbm, cols_hbm, rows_hbm, outlo_hbm, outhi_hbm,
             cols_v, rows_v, buf_a, buf_b, agg_sh, sem_a, sem_b):
        c = lax.axis_index("c")
        s = lax.axis_index("s")
        wid = s * NC + c

        pltpu.sync_copy(cols_hbm.at[wid], cols_v)
        pltpu.sync_copy(rows_hbm.at[wid], rows_v)

        def phase(h_hbm, out_hbm):
            def zrow(i, carry):
                for k in range(DH // 16):
                    buf_a[i, pl.ds(k * 16, 16)] = jnp.zeros((16,), jnp.float32)
                return carry
            lax.fori_loop(0, CHUNK, zrow, 0)
            for k in range(5):
                pltpu.sync_copy(buf_a,
                                agg_sh.at[pl.ds(s * 624 + k * CHUNK, CHUNK)])
            plsc.subcore_barrier()

            def gstart(j, buf, sem):
                pltpu.async_copy(h_hbm.at[cols_v.at[j]], buf, sem)

            def gwait(buf, sem):
                pltpu.make_async_copy(h_hbm.at[cols_v.at[0]], buf, sem).wait()

            gstart(0, buf_a, sem_a)

            def step(t, carry):
                j0 = 2 * t
                gstart(j0 + 1, buf_b, sem_b)
                gwait(buf_a, sem_a)
                pltpu.sync_copy(buf_a, agg_sh.at[rows_v.at[j0]], add=True)

                @pl.when(t < NT - 1)
                def _():
                    gstart(j0 + 2, buf_a, sem_a)

                gwait(buf_b, sem_b)
                pltpu.sync_copy(buf_b, agg_sh.at[rows_v.at[j0 + 1]], add=True)
                return carry
            lax.fori_loop(0, NT, step, 0)

            plsc.subcore_barrier()
            off = s * 624
            pltpu.sync_copy(agg_sh.at[pl.ds(off, 640)],
                            out_hbm.at[pl.ds(c * N + off, 640)])
            plsc.subcore_barrier()

        phase(hlo_hbm, outlo_hbm)
        phase(hhi_hbm, outhi_hbm)

    f = pl.kernel(
        body,
        out_type=(jax.ShapeDtypeStruct((NC * N, DH), jnp.float32),
                  jax.ShapeDtypeStruct((NC * N, DH), jnp.float32)),
        mesh=mesh,
        scratch_types=[
            pltpu.VMEM((NCHUNK, CHUNK), jnp.int32),
            pltpu.VMEM((NCHUNK, CHUNK), jnp.int32),
            pltpu.VMEM((CHUNK, DH), jnp.float32),
            pltpu.VMEM((CHUNK, DH), jnp.float32),
            pltpu.VMEM_SHARED((NP, DH), jnp.float32),
            pltpu.SemaphoreType.DMA,
            pltpu.SemaphoreType.DMA,
        ],
        compiler_params=pltpu.CompilerParams(use_tc_tiling_on_sc=False),
    )
    return f(h_lo, h_hi, cols, rows)


def _tc_layer(plo, phi, h, W, S, b):
    def body(plo0_ref, plo1_ref, phi0_ref, phi1_ref, h_ref,
             w_ref, s_ref, b_ref, olo_ref, ohi_ref):
        agg = jnp.concatenate([plo0_ref[...] + plo1_ref[...],
                               phi0_ref[...] + phi1_ref[...]], axis=1)
        acc = jnp.dot(agg, w_ref[...], preferred_element_type=jnp.float32)
        acc += jnp.dot(h_ref[...], s_ref[...], preferred_element_type=jnp.float32)
        acc += b_ref[...]
        out = jnp.maximum(acc, 0.0)
        olo_ref[...] = out[:, :DH]
        ohi_ref[...] = out[:, DH:]

    nb = N // BLK
    half = [
        pl.BlockSpec((BLK, DH), lambda i: (i, 0)),
        pl.BlockSpec((BLK, DH), lambda i: (i + nb, 0)),
    ]
    return pl.pallas_call(
        body,
        grid=(nb,),
        in_specs=half + half + [
            pl.BlockSpec((BLK, D), lambda i: (i, 0)),
            pl.BlockSpec((D, D), lambda i: (0, 0)),
            pl.BlockSpec((D, D), lambda i: (0, 0)),
            pl.BlockSpec((1, D), lambda i: (0, 0)),
        ],
        out_specs=(pl.BlockSpec((BLK, DH), lambda i: (i, 0)),
                   pl.BlockSpec((BLK, DH), lambda i: (i, 0))),
        out_shape=(jax.ShapeDtypeStruct((N, DH), jnp.float32),
                   jax.ShapeDtypeStruct((N, DH), jnp.float32)),
    )(plo, plo, phi, phi, h, W, S, b.reshape(1, D))


def _tc_layer_final(plo, phi, h_lo, h_hi, W, S, b, Wf, bf):
    def body(plo0_ref, plo1_ref, phi0_ref, phi1_ref, hlo_ref, hhi_ref,
             w_ref, s_ref, b_ref, wf_ref, bf_ref, o_ref):
        agg = jnp.concatenate([plo0_ref[...] + plo1_ref[...],
                               phi0_ref[...] + phi1_ref[...]], axis=1)
        h = jnp.concatenate([hlo_ref[...], hhi_ref[...]], axis=1)
        acc = jnp.dot(agg, w_ref[...], preferred_element_type=jnp.float32)
        acc += jnp.dot(h, s_ref[...], preferred_element_type=jnp.float32)
        acc += b_ref[...]
        t = jnp.maximum(acc, 0.0)
        out = lax.dot_general(t, wf_ref[...], (((1,), (1,)), ((), ())),
                              preferred_element_type=jnp.float32)
        o_ref[...] = out + bf_ref[...]

    nb = N // BLK
    half = [
        pl.BlockSpec((BLK, DH), lambda i: (i, 0)),
        pl.BlockSpec((BLK, DH), lambda i: (i + nb, 0)),
    ]
    return pl.pallas_call(
        body,
        grid=(nb,),
        in_specs=half + half + [
            pl.BlockSpec((BLK, DH), lambda i: (i, 0)),
            pl.BlockSpec((BLK, DH), lambda i: (i, 0)),
            pl.BlockSpec((D, D), lambda i: (0, 0)),
            pl.BlockSpec((D, D), lambda i: (0, 0)),
            pl.BlockSpec((1, D), lambda i: (0, 0)),
            pl.BlockSpec((D, D), lambda i: (0, 0)),
            pl.BlockSpec((1, D), lambda i: (0, 0)),
        ],
        out_specs=pl.BlockSpec((BLK, D), lambda i: (i, 0)),
        out_shape=jax.ShapeDtypeStruct((N, D), jnp.float32),
    )(plo, plo, phi, phi, h_lo, h_hi, W, S, b.reshape(1, D), Wf, bf.reshape(1, D))


def kernel(x, edge_index, W1, S1, b1, W2, S2, b2, Wf, bf):
    rows = edge_index[0]
    cols = edge_index[1]
    pad = E_PAD - E
    rows_p = jnp.concatenate(
        [rows, jnp.full((pad,), N, jnp.int32)]).reshape(NW, NCHUNK, CHUNK)
    cols_p = jnp.concatenate(
        [cols, jnp.zeros((pad,), jnp.int32)]).reshape(NW, NCHUNK, CHUNK)
    x_lo = x[:, :DH]
    x_hi = x[:, DH:]

    p1_lo, p1_hi = _sc_agg(x_lo, x_hi, cols_p, rows_p)
    h1_lo, h1_hi = _tc_layer(p1_lo, p1_hi, x, W1, S1, b1)
    p2_lo, p2_hi = _sc_agg(h1_lo, h1_hi, cols_p, rows_p)
    return _tc_layer_final(p2_lo, p2_hi, h1_lo, h1_hi, W2, S2, b2, Wf, bf)


# speedup vs baseline: 5.1384x; 1.5164x over previous
import jax
import jax.numpy as jnp
from jax import lax
from jax.experimental import pallas as pl
from jax.experimental.pallas import tpu as pltpu
from jax.experimental.pallas import tpu_sc as plsc

N = 10000
E = 320000
D = 128
DH = D // 2
NC = 2
NS = 16
CHUNK = 128
NCHUNK = 160
E_PAD = NS * NCHUNK * CHUNK
EPT = E_PAD // NS
NSLOT = 8
OFF = 4
NHALF = 2
HCHUNK = NCHUNK // NHALF
NWAVE = HCHUNK // NSLOT
NP = 10008
BLK = 1000


def _sc_agg(hs, cols, rows):
    mesh = plsc.VectorSubcoreMesh(core_axis_name="c", subcore_axis_name="s")

    def body(hs_hbm, cols_hbm, rows_hbm, out_hbm,
             cols_v, rows_v, bufs, agg_sh, *sems):
        gsems = sems[:NSLOT]
        ssems = sems[NSLOT:]
        c = lax.axis_index("c")
        s = lax.axis_index("s")
        h_hbm = hs_hbm.at[c]

        def zrow(i, carry):
            for k in range(DH // 16):
                bufs[0, i, pl.ds(k * 16, 16)] = jnp.zeros((16,), jnp.float32)
            return carry
        lax.fori_loop(0, CHUNK, zrow, 0)
        for k in range(5):
            pltpu.sync_copy(bufs.at[0],
                            agg_sh.at[pl.ds(s * 624 + k * CHUNK, CHUNK)])
        plsc.subcore_barrier()

        def gstart(slot, j):
            pltpu.async_copy(h_hbm.at[cols_v.at[j]], bufs.at[slot],
                             gsems[slot])

        def gwait(slot):
            pltpu.make_async_copy(h_hbm.at[cols_v.at[0]], bufs.at[slot],
                                  gsems[slot]).wait()

        def sstart(slot, j):
            pltpu.async_copy(bufs.at[slot], agg_sh.at[rows_v.at[j]],
                             ssems[slot], add=True)

        def swait(slot):
            pltpu.make_async_copy(bufs.at[slot], agg_sh.at[rows_v.at[0]],
                                  ssems[slot]).wait()

        for half in range(NHALF):
            pltpu.sync_copy(cols_hbm.at[s].at[pl.ds(half * HCHUNK, HCHUNK)],
                            cols_v)
            pltpu.sync_copy(rows_hbm.at[s].at[pl.ds(half * HCHUNK, HCHUNK)],
                            rows_v)

            for k in range(NSLOT):
                gstart(k, k)

            def wave(w, carry):
                base = w * NSLOT

                def regather(kk):
                    swait(kk)

                    @pl.when(w + 1 < NWAVE)
                    def _():
                        gstart(kk, base + NSLOT + kk)

                for k in range(NSLOT):
                    gwait(k)
                    sstart(k, base + k)
                    if k >= OFF:
                        regather(k - OFF)
                for kk in range(NSLOT - OFF, NSLOT):
                    regather(kk)
                return carry
            lax.fori_loop(0, NWAVE, wave, 0)

        plsc.subcore_barrier()
        off = s * 624
        pltpu.sync_copy(agg_sh.at[pl.ds(off, 640)],
                        out_hbm.at[c].at[pl.ds(off, 640)])

    f = pl.kernel(
        body,
        out_type=jax.ShapeDtypeStruct((NC, N, DH), jnp.float32),
        mesh=mesh,
        scratch_types=[
            pltpu.VMEM((HCHUNK, CHUNK), jnp.int32),
            pltpu.VMEM((HCHUNK, CHUNK), jnp.int32),
            pltpu.VMEM((NSLOT, CHUNK, DH), jnp.float32),
            pltpu.VMEM_SHARED((NP, DH), jnp.float32),
        ] + [pltpu.SemaphoreType.DMA] * (2 * NSLOT),
        compiler_params=pltpu.CompilerParams(use_tc_tiling_on_sc=False),
    )
    return f(hs, cols, rows)


def _tc_layer(p, h, W, S, b):
    def body(p_ref, h_ref, w_ref, s_ref, b_ref, o_ref):
        agg = jnp.concatenate([p_ref[0], p_ref[1]], axis=1)
        acc = jnp.dot(agg, w_ref[...], preferred_element_type=jnp.float32)
        acc += jnp.dot(h_ref[...], s_ref[...], preferred_element_type=jnp.float32)
        acc += b_ref[...]
        out = jnp.maximum(acc, 0.0)
        o_ref[0] = out[:, :DH]
        o_ref[1] = out[:, DH:]

    nb = N // BLK
    return pl.pallas_call(
        body,
        grid=(nb,),
        in_specs=[
            pl.BlockSpec((NC, BLK, DH), lambda i: (0, i, 0)),
            pl.BlockSpec((BLK, D), lambda i: (i, 0)),
            pl.BlockSpec((D, D), lambda i: (0, 0)),
            pl.BlockSpec((D, D), lambda i: (0, 0)),
            pl.BlockSpec((1, D), lambda i: (0, 0)),
        ],
        out_specs=pl.BlockSpec((NC, BLK, DH), lambda i: (0, i, 0)),
        out_shape=jax.ShapeDtypeStruct((NC, N, DH), jnp.float32),
    )(p, h, W, S, b.reshape(1, D))


def _tc_layer_final(p, hs, W, S, b, Wf, bf):
    def body(p_ref, hs_ref, w_ref, s_ref, b_ref, wf_ref, bf_ref, o_ref):
        agg = jnp.concatenate([p_ref[0], p_ref[1]], axis=1)
        h = jnp.concatenate([hs_ref[0], hs_ref[1]], axis=1)
        acc = jnp.dot(agg, w_ref[...], preferred_element_type=jnp.float32)
        acc += jnp.dot(h, s_ref[...], preferred_element_type=jnp.float32)
        acc += b_ref[...]
        t = jnp.maximum(acc, 0.0)
        out = lax.dot_general(t, wf_ref[...], (((1,), (1,)), ((), ())),
                              preferred_element_type=jnp.float32)
        o_ref[...] = out + bf_ref[...]

    nb = N // BLK
    return pl.pallas_call(
        body,
        grid=(nb,),
        in_specs=[
            pl.BlockSpec((NC, BLK, DH), lambda i: (0, i, 0)),
            pl.BlockSpec((NC, BLK, DH), lambda i: (0, i, 0)),
            pl.BlockSpec((D, D), lambda i: (0, 0)),
            pl.BlockSpec((D, D), lambda i: (0, 0)),
            pl.BlockSpec((1, D), lambda i: (0, 0)),
            pl.BlockSpec((D, D), lambda i: (0, 0)),
            pl.BlockSpec((1, D), lambda i: (0, 0)),
        ],
        out_specs=pl.BlockSpec((BLK, D), lambda i: (i, 0)),
        out_shape=jax.ShapeDtypeStruct((N, D), jnp.float32),
    )(p, hs, W, S, b.reshape(1, D), Wf, bf.reshape(1, D))


def kernel(x, edge_index, W1, S1, b1, W2, S2, b2, Wf, bf):
    rows = edge_index[0]
    cols = edge_index[1]
    pad = E_PAD - E
    rows_p = jnp.concatenate(
        [rows, jnp.full((pad,), N, jnp.int32)]).reshape(NS, NCHUNK, CHUNK)
    cols_p = jnp.concatenate(
        [cols, jnp.zeros((pad,), jnp.int32)]).reshape(NS, NCHUNK, CHUNK)
    xs = jnp.stack([x[:, :DH], x[:, DH:]])

    p1 = _sc_agg(xs, cols_p, rows_p)
    h1s = _tc_layer(p1, x, W1, S1, b1)
    p2 = _sc_agg(h1s, cols_p, rows_p)
    return _tc_layer_final(p2, h1s, W2, S2, b2, Wf, bf)


# speedup vs baseline: 9.2410x; 1.7984x over previous
import jax
import jax.numpy as jnp
from jax import lax
from jax.experimental import pallas as pl
from jax.experimental.pallas import tpu as pltpu
from jax.experimental.pallas import tpu_sc as plsc

N = 10000
E = 320000
D = 128
DH = D // 2
NC = 2
NS = 16
CHUNK = 64
NCHUNK = 320
E_PAD = NS * NCHUNK * CHUNK
EPT = E_PAD // NS
NSLOT = 8
OFF = 4
NHALF = 4
HCHUNK = NCHUNK // NHALF
NWAVE = HCHUNK // NSLOT
NP = 10008
BLK = 1000


def _sc_agg(hs, cols, rows):
    mesh = plsc.VectorSubcoreMesh(core_axis_name="c", subcore_axis_name="s")

    def body(hs_hbm, cols_hbm, rows_hbm, out_hbm,
             cols_v, rows_v, bufs, h_sh, agg_sh, *sems):
        gsems = sems[:NSLOT]
        ssems = sems[NSLOT:]
        c = lax.axis_index("c")
        s = lax.axis_index("s")
        h_hbm = hs_hbm.at[c]
        off = s * 624

        pltpu.sync_copy(h_hbm.at[pl.ds(off, 640)], h_sh.at[pl.ds(off, 640)])

        def zrow(i, carry):
            for k in range(DH // 16):
                bufs[0, i, pl.ds(k * 16, 16)] = jnp.zeros((16,), jnp.float32)
            return carry
        lax.fori_loop(0, CHUNK, zrow, 0)
        for k in range(10):
            pltpu.sync_copy(bufs.at[0],
                            agg_sh.at[pl.ds(off + k * CHUNK, CHUNK)])
        plsc.subcore_barrier()

        def gstart(slot, j):
            pltpu.async_copy(h_sh.at[cols_v.at[j]], bufs.at[slot],
                             gsems[slot])

        def gwait(slot):
            pltpu.make_async_copy(h_sh.at[cols_v.at[0]], bufs.at[slot],
                                  gsems[slot]).wait()

        def sstart(slot, j):
            pltpu.async_copy(bufs.at[slot], agg_sh.at[rows_v.at[j]],
                             ssems[slot], add=True)

        def swait(slot):
            pltpu.make_async_copy(bufs.at[slot], agg_sh.at[rows_v.at[0]],
                                  ssems[slot]).wait()

        for half in range(NHALF):
            pltpu.sync_copy(cols_hbm.at[s].at[pl.ds(half * HCHUNK, HCHUNK)],
                            cols_v)
            pltpu.sync_copy(rows_hbm.at[s].at[pl.ds(half * HCHUNK, HCHUNK)],
                            rows_v)

            for k in range(NSLOT):
                gstart(k, k)

            def wave(w, carry):
                base = w * NSLOT

                def regather(kk):
                    swait(kk)

                    @pl.when(w + 1 < NWAVE)
                    def _():
                        gstart(kk, base + NSLOT + kk)

                for k in range(NSLOT):
                    gwait(k)
                    sstart(k, base + k)
                    if k >= OFF:
                        regather(k - OFF)
                for kk in range(NSLOT - OFF, NSLOT):
                    regather(kk)
                return carry
            lax.fori_loop(0, NWAVE, wave, 0)

        plsc.subcore_barrier()
        off = s * 624
        pltpu.sync_copy(agg_sh.at[pl.ds(off, 640)],
                        out_hbm.at[c].at[pl.ds(off, 640)])

    f = pl.kernel(
        body,
        out_type=jax.ShapeDtypeStruct((NC, N, DH), jnp.float32),
        mesh=mesh,
        scratch_types=[
            pltpu.VMEM((HCHUNK, CHUNK), jnp.int32),
            pltpu.VMEM((HCHUNK, CHUNK), jnp.int32),
            pltpu.VMEM((NSLOT, CHUNK, DH), jnp.float32),
            pltpu.VMEM_SHARED((N, DH), jnp.float32),
            pltpu.VMEM_SHARED((NP, DH), jnp.float32),
        ] + [pltpu.SemaphoreType.DMA] * (2 * NSLOT),
        compiler_params=pltpu.CompilerParams(use_tc_tiling_on_sc=False),
    )
    return f(hs, cols, rows)


def _tc_layer(p, h, W, S, b):
    def body(p_ref, h_ref, w_ref, s_ref, b_ref, o_ref):
        agg = jnp.concatenate([p_ref[0], p_ref[1]], axis=1)
        acc = jnp.dot(agg, w_ref[...], preferred_element_type=jnp.float32)
        acc += jnp.dot(h_ref[...], s_ref[...], preferred_element_type=jnp.float32)
        acc += b_ref[...]
        out = jnp.maximum(acc, 0.0)
        o_ref[0] = out[:, :DH]
        o_ref[1] = out[:, DH:]

    nb = N // BLK
    return pl.pallas_call(
        body,
        grid=(nb,),
        in_specs=[
            pl.BlockSpec((NC, BLK, DH), lambda i: (0, i, 0)),
            pl.BlockSpec((BLK, D), lambda i: (i, 0)),
            pl.BlockSpec((D, D), lambda i: (0, 0)),
            pl.BlockSpec((D, D), lambda i: (0, 0)),
            pl.BlockSpec((1, D), lambda i: (0, 0)),
        ],
        out_specs=pl.BlockSpec((NC, BLK, DH), lambda i: (0, i, 0)),
        out_shape=jax.ShapeDtypeStruct((NC, N, DH), jnp.float32),
    )(p, h, W, S, b.reshape(1, D))


def _tc_layer_final(p, hs, W, S, b, Wf, bf):
    def body(p_ref, hs_ref, w_ref, s_ref, b_ref, wf_ref, bf_ref, o_ref):
        agg = jnp.concatenate([p_ref[0], p_ref[1]], axis=1)
        h = jnp.concatenate([hs_ref[0], hs_ref[1]], axis=1)
        acc = jnp.dot(agg, w_ref[...], preferred_element_type=jnp.float32)
        acc += jnp.dot(h, s_ref[...], preferred_element_type=jnp.float32)
        acc += b_ref[...]
        t = jnp.maximum(acc, 0.0)
        out = lax.dot_general(t, wf_ref[...], (((1,), (1,)), ((), ())),
                              preferred_element_type=jnp.float32)
        o_ref[...] = out + bf_ref[...]

    nb = N // BLK
    return pl.pallas_call(
        body,
        grid=(nb,),
        in_specs=[
            pl.BlockSpec((NC, BLK, DH), lambda i: (0, i, 0)),
            pl.BlockSpec((NC, BLK, DH), lambda i: (0, i, 0)),
            pl.BlockSpec((D, D), lambda i: (0, 0)),
            pl.BlockSpec((D, D), lambda i: (0, 0)),
            pl.BlockSpec((1, D), lambda i: (0, 0)),
            pl.BlockSpec((D, D), lambda i: (0, 0)),
            pl.BlockSpec((1, D), lambda i: (0, 0)),
        ],
        out_specs=pl.BlockSpec((BLK, D), lambda i: (i, 0)),
        out_shape=jax.ShapeDtypeStruct((N, D), jnp.float32),
    )(p, hs, W, S, b.reshape(1, D), Wf, bf.reshape(1, D))


def kernel(x, edge_index, W1, S1, b1, W2, S2, b2, Wf, bf):
    rows = edge_index[0]
    cols = edge_index[1]
    pad = E_PAD - E
    rows_p = jnp.concatenate(
        [rows, jnp.full((pad,), N, jnp.int32)]).reshape(NS, NCHUNK, CHUNK)
    cols_p = jnp.concatenate(
        [cols, jnp.zeros((pad,), jnp.int32)]).reshape(NS, NCHUNK, CHUNK)
    xs = jnp.stack([x[:, :DH], x[:, DH:]])

    p1 = _sc_agg(xs, cols_p, rows_p)
    h1s = _tc_layer(p1, x, W1, S1, b1)
    p2 = _sc_agg(h1s, cols_p, rows_p)
    return _tc_layer_final(p2, h1s, W2, S2, b2, Wf, bf)


# speedup vs baseline: 10.6517x; 1.1526x over previous
import jax
import jax.numpy as jnp
from jax import lax
from jax.experimental import pallas as pl
from jax.experimental.pallas import tpu as pltpu
from jax.experimental.pallas import tpu_sc as plsc

N = 10000
E = 320000
D = 128
DH = D // 2
NC = 2
NS = 16
CHUNK = 80
NCHUNK = 250
EPT = E // NS
NSLOT = 5
OFF = 2
NHALF = 2
HCHUNK = NCHUNK // NHALF
NWAVE = HCHUNK // NSLOT
BLK = 1000


def _sc_agg(h, cols, rows):
    mesh = plsc.VectorSubcoreMesh(core_axis_name="c", subcore_axis_name="s")

    def body(h_hbm, cols_hbm, rows_hbm, out_hbm,
             cols_v, rows_v, bufs, h_sh, agg_sh, *sems):
        gsems = sems[:NSLOT]
        ssems = sems[NSLOT:]
        c = lax.axis_index("c")
        s = lax.axis_index("s")
        off = s * 624

        pltpu.sync_copy(h_hbm.at[pl.ds(off, 640), pl.ds(c * DH, DH)],
                        h_sh.at[pl.ds(off, 640)])

        def zrow(i, carry):
            for k in range(DH // 16):
                bufs[0, i, pl.ds(k * 16, 16)] = jnp.zeros((16,), jnp.float32)
            return carry
        lax.fori_loop(0, CHUNK, zrow, 0)
        for k in range(8):
            pltpu.sync_copy(bufs.at[0],
                            agg_sh.at[pl.ds(off + k * CHUNK, CHUNK)])
        plsc.subcore_barrier()

        def gstart(slot, j):
            pltpu.async_copy(h_sh.at[cols_v.at[j]], bufs.at[slot],
                             gsems[slot])

        def gwait(slot):
            pltpu.make_async_copy(h_sh.at[cols_v.at[0]], bufs.at[slot],
                                  gsems[slot]).wait()

        def sstart(slot, j):
            pltpu.async_copy(bufs.at[slot], agg_sh.at[rows_v.at[j]],
                             ssems[slot], add=True)

        def swait(slot):
            pltpu.make_async_copy(bufs.at[slot], agg_sh.at[rows_v.at[0]],
                                  ssems[slot]).wait()

        for half in range(NHALF):
            pltpu.sync_copy(cols_hbm.at[s].at[pl.ds(half * HCHUNK, HCHUNK)],
                            cols_v)
            pltpu.sync_copy(rows_hbm.at[s].at[pl.ds(half * HCHUNK, HCHUNK)],
                            rows_v)

            for k in range(NSLOT):
                gstart(k, k)

            def wave(w, carry):
                base = w * NSLOT

                def regather(kk):
                    swait(kk)

                    @pl.when(w + 1 < NWAVE)
                    def _():
                        gstart(kk, base + NSLOT + kk)

                for k in range(NSLOT):
                    gwait(k)
                    sstart(k, base + k)
                    if k >= OFF:
                        regather(k - OFF)
                for kk in range(NSLOT - OFF, NSLOT):
                    regather(kk)
                return carry
            lax.fori_loop(0, NWAVE, wave, 0)

        plsc.subcore_barrier()
        pltpu.sync_copy(agg_sh.at[pl.ds(off, 640)],
                        out_hbm.at[c].at[pl.ds(off, 640)])

    f = pl.kernel(
        body,
        out_type=jax.ShapeDtypeStruct((NC, N, DH), jnp.float32),
        mesh=mesh,
        scratch_types=[
            pltpu.VMEM((HCHUNK, CHUNK), jnp.int32),
            pltpu.VMEM((HCHUNK, CHUNK), jnp.int32),
            pltpu.VMEM((NSLOT, CHUNK, DH), jnp.float32),
            pltpu.VMEM_SHARED((N, DH), jnp.float32),
            pltpu.VMEM_SHARED((N, DH), jnp.float32),
        ] + [pltpu.SemaphoreType.DMA] * (2 * NSLOT),
        compiler_params=pltpu.CompilerParams(use_tc_tiling_on_sc=False),
    )
    return f(h, cols, rows)


def _tc_layer(p, h, W, S, b):
    def body(p_ref, h_ref, w_ref, s_ref, b_ref, o_ref):
        agg = jnp.concatenate([p_ref[0], p_ref[1]], axis=1)
        acc = jnp.dot(agg, w_ref[...], preferred_element_type=jnp.float32)
        acc += jnp.dot(h_ref[...], s_ref[...], preferred_element_type=jnp.float32)
        acc += b_ref[...]
        o_ref[...] = jnp.maximum(acc, 0.0)

    nb = N // BLK
    return pl.pallas_call(
        body,
        grid=(nb,),
        in_specs=[
            pl.BlockSpec((NC, BLK, DH), lambda i: (0, i, 0)),
            pl.BlockSpec((BLK, D), lambda i: (i, 0)),
            pl.BlockSpec((D, D), lambda i: (0, 0)),
            pl.BlockSpec((D, D), lambda i: (0, 0)),
            pl.BlockSpec((1, D), lambda i: (0, 0)),
        ],
        out_specs=pl.BlockSpec((BLK, D), lambda i: (i, 0)),
        out_shape=jax.ShapeDtypeStruct((N, D), jnp.float32),
    )(p, h, W, S, b.reshape(1, D))


def _tc_layer_final(p, h, W, S, b, Wf, bf):
    def body(p_ref, h_ref, w_ref, s_ref, b_ref, wf_ref, bf_ref, o_ref):
        agg = jnp.concatenate([p_ref[0], p_ref[1]], axis=1)
        acc = jnp.dot(agg, w_ref[...], preferred_element_type=jnp.float32)
        acc += jnp.dot(h_ref[...], s_ref[...], preferred_element_type=jnp.float32)
        acc += b_ref[...]
        t = jnp.maximum(acc, 0.0)
        out = lax.dot_general(t, wf_ref[...], (((1,), (1,)), ((), ())),
                              preferred_element_type=jnp.float32)
        o_ref[...] = out + bf_ref[...]

    nb = N // BLK
    return pl.pallas_call(
        body,
        grid=(nb,),
        in_specs=[
            pl.BlockSpec((NC, BLK, DH), lambda i: (0, i, 0)),
            pl.BlockSpec((BLK, D), lambda i: (i, 0)),
            pl.BlockSpec((D, D), lambda i: (0, 0)),
            pl.BlockSpec((D, D), lambda i: (0, 0)),
            pl.BlockSpec((1, D), lambda i: (0, 0)),
            pl.BlockSpec((D, D), lambda i: (0, 0)),
            pl.BlockSpec((1, D), lambda i: (0, 0)),
        ],
        out_specs=pl.BlockSpec((BLK, D), lambda i: (i, 0)),
        out_shape=jax.ShapeDtypeStruct((N, D), jnp.float32),
    )(p, h, W, S, b.reshape(1, D), Wf, bf.reshape(1, D))


def kernel(x, edge_index, W1, S1, b1, W2, S2, b2, Wf, bf):
    rows_p = edge_index[0].reshape(NS, NCHUNK, CHUNK)
    cols_p = edge_index[1].reshape(NS, NCHUNK, CHUNK)

    p1 = _sc_agg(x, cols_p, rows_p)
    h1 = _tc_layer(p1, x, W1, S1, b1)
    p2 = _sc_agg(h1, cols_p, rows_p)
    return _tc_layer_final(p2, h1, W2, S2, b2, Wf, bf)


# speedup vs baseline: 10.8496x; 1.0186x over previous
import jax
import jax.numpy as jnp
from jax import lax
from jax.experimental import pallas as pl
from jax.experimental.pallas import tpu as pltpu
from jax.experimental.pallas import tpu_sc as plsc

N = 10000
E = 320000
D = 128
DH = D // 2
NC = 2
NS = 16
CHUNK = 80
NCHUNK = 250
EPT = E // NS
NSLOT = 5
OFF = 2
NHALF = 2
HCHUNK = NCHUNK // NHALF
NWAVE = HCHUNK // NSLOT
BLK = 2000


def _sc_agg(h, cols, rows):
    mesh = plsc.VectorSubcoreMesh(core_axis_name="c", subcore_axis_name="s")

    def body(h_hbm, cols_hbm, rows_hbm, out_hbm,
             cols_v, rows_v, bufs, h_sh, agg_sh, *sems):
        gsems = sems[:NSLOT]
        ssems = sems[NSLOT:]
        c = lax.axis_index("c")
        s = lax.axis_index("s")
        off = s * 624

        pltpu.sync_copy(h_hbm.at[pl.ds(off, 640), pl.ds(c * DH, DH)],
                        h_sh.at[pl.ds(off, 640)])

        def zrow(i, carry):
            for k in range(DH // 16):
                bufs[0, i, pl.ds(k * 16, 16)] = jnp.zeros((16,), jnp.float32)
            return carry
        lax.fori_loop(0, CHUNK, zrow, 0)
        for k in range(8):
            pltpu.sync_copy(bufs.at[0],
                            agg_sh.at[pl.ds(off + k * CHUNK, CHUNK)])
        plsc.subcore_barrier()

        def gstart(slot, j):
            pltpu.async_copy(h_sh.at[cols_v.at[pl.ds(j * CHUNK, CHUNK)]],
                             bufs.at[slot], gsems[slot])

        def gwait(slot):
            pltpu.make_async_copy(h_sh.at[cols_v.at[pl.ds(0, CHUNK)]],
                                  bufs.at[slot], gsems[slot]).wait()

        def sstart(slot, j):
            pltpu.async_copy(bufs.at[slot],
                             agg_sh.at[rows_v.at[pl.ds(j * CHUNK, CHUNK)]],
                             ssems[slot], add=True)

        def swait(slot):
            pltpu.make_async_copy(bufs.at[slot],
                                  agg_sh.at[rows_v.at[pl.ds(0, CHUNK)]],
                                  ssems[slot]).wait()

        for half in range(NHALF):
            gbase = s * EPT + half * (HCHUNK * CHUNK)
            pltpu.sync_copy(cols_hbm.at[pl.ds(gbase, HCHUNK * CHUNK)], cols_v)
            pltpu.sync_copy(rows_hbm.at[pl.ds(gbase, HCHUNK * CHUNK)], rows_v)

            for k in range(NSLOT):
                gstart(k, k)

            def wave(w, carry):
                base = w * NSLOT

                def regather(kk):
                    swait(kk)

                    @pl.when(w + 1 < NWAVE)
                    def _():
                        gstart(kk, base + NSLOT + kk)

                for k in range(NSLOT):
                    gwait(k)
                    sstart(k, base + k)
                    if k >= OFF:
                        regather(k - OFF)
                for kk in range(NSLOT - OFF, NSLOT):
                    regather(kk)
                return carry
            lax.fori_loop(0, NWAVE, wave, 0)

        plsc.subcore_barrier()
        pltpu.sync_copy(agg_sh.at[pl.ds(off, 640)],
                        out_hbm.at[c].at[pl.ds(off, 640)])

    f = pl.kernel(
        body,
        out_type=jax.ShapeDtypeStruct((NC, N, DH), jnp.float32),
        mesh=mesh,
        scratch_types=[
            pltpu.VMEM((HCHUNK * CHUNK,), jnp.int32),
            pltpu.VMEM((HCHUNK * CHUNK,), jnp.int32),
            pltpu.VMEM((NSLOT, CHUNK, DH), jnp.float32),
            pltpu.VMEM_SHARED((N, DH), jnp.float32),
            pltpu.VMEM_SHARED((N, DH), jnp.float32),
        ] + [pltpu.SemaphoreType.DMA] * (2 * NSLOT),
        compiler_params=pltpu.CompilerParams(use_tc_tiling_on_sc=False),
    )
    return f(h, cols, rows)


def _tc_layer(p, h, W, S, b):
    def body(p_ref, h_ref, w_ref, s_ref, b_ref, o_ref):
        agg = jnp.concatenate([p_ref[0], p_ref[1]], axis=1)
        acc = jnp.dot(agg, w_ref[...], preferred_element_type=jnp.float32)
        acc += jnp.dot(h_ref[...], s_ref[...], preferred_element_type=jnp.float32)
        acc += b_ref[...]
        o_ref[...] = jnp.maximum(acc, 0.0)

    nb = N // BLK
    return pl.pallas_call(
        body,
        grid=(nb,),
        in_specs=[
            pl.BlockSpec((NC, BLK, DH), lambda i: (0, i, 0)),
            pl.BlockSpec((BLK, D), lambda i: (i, 0)),
            pl.BlockSpec((D, D), lambda i: (0, 0)),
            pl.BlockSpec((D, D), lambda i: (0, 0)),
            pl.BlockSpec((1, D), lambda i: (0, 0)),
        ],
        out_specs=pl.BlockSpec((BLK, D), lambda i: (i, 0)),
        out_shape=jax.ShapeDtypeStruct((N, D), jnp.float32),
    )(p, h, W, S, b.reshape(1, D))


def _tc_layer_final(p, h, W, S, b, Wf, bf):
    def body(p_ref, h_ref, w_ref, s_ref, b_ref, wf_ref, bf_ref, o_ref):
        agg = jnp.concatenate([p_ref[0], p_ref[1]], axis=1)
        acc = jnp.dot(agg, w_ref[...], preferred_element_type=jnp.float32)
        acc += jnp.dot(h_ref[...], s_ref[...], preferred_element_type=jnp.float32)
        acc += b_ref[...]
        t = jnp.maximum(acc, 0.0)
        out = lax.dot_general(t, wf_ref[...], (((1,), (1,)), ((), ())),
                              preferred_element_type=jnp.float32)
        o_ref[...] = out + bf_ref[...]

    nb = N // BLK
    return pl.pallas_call(
        body,
        grid=(nb,),
        in_specs=[
            pl.BlockSpec((NC, BLK, DH), lambda i: (0, i, 0)),
            pl.BlockSpec((BLK, D), lambda i: (i, 0)),
            pl.BlockSpec((D, D), lambda i: (0, 0)),
            pl.BlockSpec((D, D), lambda i: (0, 0)),
            pl.BlockSpec((1, D), lambda i: (0, 0)),
            pl.BlockSpec((D, D), lambda i: (0, 0)),
            pl.BlockSpec((1, D), lambda i: (0, 0)),
        ],
        out_specs=pl.BlockSpec((BLK, D), lambda i: (i, 0)),
        out_shape=jax.ShapeDtypeStruct((N, D), jnp.float32),
    )(p, h, W, S, b.reshape(1, D), Wf, bf.reshape(1, D))


def kernel(x, edge_index, W1, S1, b1, W2, S2, b2, Wf, bf):
    rows_p = edge_index[0]
    cols_p = edge_index[1]

    p1 = _sc_agg(x, cols_p, rows_p)
    h1 = _tc_layer(p1, x, W1, S1, b1)
    p2 = _sc_agg(h1, cols_p, rows_p)
    return _tc_layer_final(p2, h1, W2, S2, b2, Wf, bf)


# speedup vs baseline: 11.2307x; 1.0351x over previous
import jax
import jax.numpy as jnp
from jax import lax
from jax.experimental import pallas as pl
from jax.experimental.pallas import tpu as pltpu
from jax.experimental.pallas import tpu_sc as plsc

N = 10000
E = 320000
D = 128
DH = D // 2
NC = 2
NS = 16
CHUNK = 80
NCHUNK = 250
EPT = E // NS
NSLOT = 5
OFF = 2
NHALF = 2
HCHUNK = NCHUNK // NHALF
NWAVE = HCHUNK // NSLOT
BLK = 2000


def _sc_agg(h, cols, rows):
    mesh = plsc.VectorSubcoreMesh(core_axis_name="c", subcore_axis_name="s")

    def body(h_hbm, cols_hbm, rows_hbm, out_hbm,
             cols_v, rows_v, bufs, h_sh, agg_sh, *sems):
        gsems = sems[:NSLOT]
        ssems = sems[NSLOT:]
        c = lax.axis_index("c")
        s = lax.axis_index("s")
        off = s * 624

        pltpu.sync_copy(h_hbm.at[pl.ds(off, 640), pl.ds(c * DH, DH)],
                        h_sh.at[pl.ds(off, 640)])

        def zrow(i, carry):
            for k in range(DH // 16):
                bufs[0, i, pl.ds(k * 16, 16)] = jnp.zeros((16,), jnp.float32)
            return carry
        lax.fori_loop(0, CHUNK, zrow, 0)
        for k in range(8):
            pltpu.sync_copy(bufs.at[0],
                            agg_sh.at[pl.ds(off + k * CHUNK, CHUNK)])
        plsc.subcore_barrier()

        def gstart(slot, j):
            pltpu.async_copy(h_sh.at[cols_v.at[pl.ds(j * CHUNK, CHUNK)]],
                             bufs.at[slot], gsems[slot])

        def gwait(slot):
            pltpu.make_async_copy(h_sh.at[cols_v.at[pl.ds(0, CHUNK)]],
                                  bufs.at[slot], gsems[slot]).wait()

        def sstart(slot, j):
            pltpu.async_copy(bufs.at[slot],
                             agg_sh.at[rows_v.at[pl.ds(j * CHUNK, CHUNK)]],
                             ssems[slot], add=True)

        def swait(slot):
            pltpu.make_async_copy(bufs.at[slot],
                                  agg_sh.at[rows_v.at[pl.ds(0, CHUNK)]],
                                  ssems[slot]).wait()

        for half in range(NHALF):
            gbase = s * EPT + half * (HCHUNK * CHUNK)
            pltpu.sync_copy(cols_hbm.at[pl.ds(gbase, HCHUNK * CHUNK)], cols_v)
            pltpu.sync_copy(rows_hbm.at[pl.ds(gbase, HCHUNK * CHUNK)], rows_v)

            for k in range(NSLOT):
                gstart(k, k)

            def wave(w, carry):
                base = w * NSLOT

                def regather(kk):
                    swait(kk)

                    @pl.when(w + 1 < NWAVE)
                    def _():
                        gstart(kk, base + NSLOT + kk)

                for k in range(NSLOT):
                    gwait(k)
                    sstart(k, base + k)
                    if k >= OFF:
                        regather(k - OFF)
                for kk in range(NSLOT - OFF, NSLOT):
                    regather(kk)
                return carry
            lax.fori_loop(0, NWAVE, wave, 0)

        plsc.subcore_barrier()
        pltpu.sync_copy(agg_sh.at[pl.ds(off, 640)],
                        out_hbm.at[c].at[pl.ds(off, 640)])

    f = pl.kernel(
        body,
        out_type=jax.ShapeDtypeStruct((NC, N, DH), jnp.float32),
        mesh=mesh,
        scratch_types=[
            pltpu.VMEM((HCHUNK * CHUNK,), jnp.int32),
            pltpu.VMEM((HCHUNK * CHUNK,), jnp.int32),
            pltpu.VMEM((NSLOT, CHUNK, DH), jnp.float32),
            pltpu.VMEM_SHARED((N, DH), jnp.float32),
            pltpu.VMEM_SHARED((N, DH), jnp.float32),
        ] + [pltpu.SemaphoreType.DMA] * (2 * NSLOT),
        compiler_params=pltpu.CompilerParams(use_tc_tiling_on_sc=False),
    )
    return f(h, cols, rows)


def _tc_layer(p, h, W, S, b):
    def body(p_ref, h_ref, w_ref, s_ref, b_ref, o_ref):
        agg = jnp.concatenate([p_ref[0], p_ref[1]], axis=1)
        acc = jnp.dot(agg, w_ref[...], preferred_element_type=jnp.float32)
        acc += jnp.dot(h_ref[...], s_ref[...], preferred_element_type=jnp.float32)
        acc += b_ref[...]
        o_ref[...] = jnp.maximum(acc, 0.0)

    nb = N // BLK
    return pl.pallas_call(
        body,
        grid=(nb,),
        in_specs=[
            pl.BlockSpec((NC, BLK, DH), lambda i: (0, i, 0)),
            pl.BlockSpec((BLK, D), lambda i: (i, 0)),
            pl.BlockSpec((D, D), lambda i: (0, 0)),
            pl.BlockSpec((D, D), lambda i: (0, 0)),
            pl.BlockSpec((1, D), lambda i: (0, 0)),
        ],
        out_specs=pl.BlockSpec((BLK, D), lambda i: (i, 0)),
        out_shape=jax.ShapeDtypeStruct((N, D), jnp.float32),
    )(p, h, W, S, b.reshape(1, D))


def _tc_layer_final(p, h, W, S, b, Wf, bf):
    def body(p_ref, h_ref, w_ref, s_ref, b_ref, wf_ref, bf_ref, o_ref):
        agg = jnp.concatenate([p_ref[0], p_ref[1]], axis=1)
        acc = jnp.dot(agg, w_ref[...], preferred_element_type=jnp.float32)
        acc += jnp.dot(h_ref[...], s_ref[...], preferred_element_type=jnp.float32)
        acc += b_ref[...]
        t = jnp.maximum(acc, 0.0)
        out = lax.dot_general(t, wf_ref[...], (((1,), (1,)), ((), ())),
                              preferred_element_type=jnp.float32)
        o_ref[...] = out + bf_ref[...]

    nb = N // BLK
    return pl.pallas_call(
        body,
        grid=(nb,),
        in_specs=[
            pl.BlockSpec((NC, BLK, DH), lambda i: (0, i, 0)),
            pl.BlockSpec((BLK, D), lambda i: (i, 0)),
            pl.BlockSpec((D, D), lambda i: (0, 0)),
            pl.BlockSpec((D, D), lambda i: (0, 0)),
            pl.BlockSpec((1, D), lambda i: (0, 0)),
            pl.BlockSpec((D, D), lambda i: (0, 0)),
            pl.BlockSpec((1, D), lambda i: (0, 0)),
        ],
        out_specs=pl.BlockSpec((BLK, D), lambda i: (i, 0)),
        out_shape=jax.ShapeDtypeStruct((N, D), jnp.float32),
    )(p, h, W, S, b.reshape(1, D), Wf, bf.reshape(1, D))


def _tc_split_edges(edge_index):
    def body(e_ref, r_ref, c_ref):
        r_ref[...] = e_ref[0].reshape(r_ref.shape)
        c_ref[...] = e_ref[1].reshape(c_ref.shape)

    return pl.pallas_call(
        body,
        out_shape=(jax.ShapeDtypeStruct((E // 128, 128), jnp.int32),
                   jax.ShapeDtypeStruct((E // 128, 128), jnp.int32)),
    )(edge_index)


def kernel(x, edge_index, W1, S1, b1, W2, S2, b2, Wf, bf):
    rows_2d, cols_2d = _tc_split_edges(edge_index)
    rows_p = rows_2d.reshape(E)
    cols_p = cols_2d.reshape(E)

    p1 = _sc_agg(x, cols_p, rows_p)
    h1 = _tc_layer(p1, x, W1, S1, b1)
    p2 = _sc_agg(h1, cols_p, rows_p)
    return _tc_layer_final(p2, h1, W2, S2, b2, Wf, bf)
